# baseline (speedup 1.0000x reference)
import functools

import jax
import jax.numpy as jnp
from jax import lax
from jax.experimental import pallas as pl
from jax.experimental.pallas import tpu as pltpu
from jax.experimental.pallas import tpu_sc as plsc

B, C, H, W = 32, 128, 100, 100
HW = H * W
V = B * HW
N = 65536
NC, NS, L = 2, 16, 16
NW = NC * NS
PER_W = N // NW
CHUNK = 128
NCHUNK = PER_W // CHUNK
CI = C // 2


def _transpose_body(x_ref, o_ref):
    x = x_ref[0]
    o_ref[0] = jnp.transpose(x, (1, 2, 0)).reshape(HW, C)


def _build_table(fused_grid):
    out = pl.pallas_call(
        _transpose_body,
        grid=(B,),
        in_specs=[pl.BlockSpec((1, C, H, W), lambda b: (b, 0, 0, 0))],
        out_specs=pl.BlockSpec((1, HW, C), lambda b: (b, 0, 0)),
        out_shape=jax.ShapeDtypeStruct((B, HW, C), jnp.float32),
    )(fused_grid)
    return out.reshape(V, C)


_sc_mesh = plsc.VectorSubcoreMesh(core_axis_name="c", subcore_axis_name="s")


@functools.partial(
    pl.kernel,
    mesh=_sc_mesh,
    out_type=jax.ShapeDtypeStruct((N, C), jnp.float32),
    scratch_types=[
        pltpu.VMEM((PER_W,), jnp.int32),
        pltpu.VMEM((CHUNK, C), jnp.float32),
        pltpu.SemaphoreType.DMA,
    ],
)
def _sc_gather(table_hbm, idx_hbm, out_hbm, idx_v, rows_v, gsem):
    wid = lax.axis_index("c") * NS + lax.axis_index("s")
    base = wid * PER_W
    pltpu.sync_copy(idx_hbm.at[pl.ds(base, PER_W)], idx_v)

    @pl.loop(0, NCHUNK)
    def _chunk(k):
        off = base + k * CHUNK
        pltpu.async_copy(
            table_hbm.at[idx_v.at[pl.ds(k * CHUNK, CHUNK)]], rows_v,
            gsem).wait()
        pltpu.sync_copy(rows_v, out_hbm.at[pl.ds(off, CHUNK)])


def _add_body(g_ref, a_ref, o_ref):
    o_ref[...] = g_ref[...] + a_ref[...]


def _tc_add(g, a):
    blk = 4096
    return pl.pallas_call(
        _add_body,
        grid=(N // blk,),
        in_specs=[pl.BlockSpec((blk, C), lambda i: (i, 0)),
                  pl.BlockSpec((blk, C), lambda i: (i, 0))],
        out_specs=pl.BlockSpec((blk, C), lambda i: (i, 0)),
        out_shape=jax.ShapeDtypeStruct((N, C), jnp.float32),
    )(g, a)


def _stream_body(x_ref, o_ref):
    o_ref[...] = x_ref[...] + 1.0


def kernel(fused_grid, agent_encodings, fetch_coordinates):
    return pl.pallas_call(
        _stream_body,
        grid=(B,),
        in_specs=[pl.BlockSpec((1, C, H, W), lambda b: (b, 0, 0, 0))],
        out_specs=pl.BlockSpec((1, C, H, W), lambda b: (b, 0, 0, 0)),
        out_shape=jax.ShapeDtypeStruct((B, C, H, W), jnp.float32),
    )(fused_grid)


# speedup vs baseline: 1.1315x; 1.1315x over previous
import functools

import jax
import jax.numpy as jnp
from jax import lax
from jax.experimental import pallas as pl
from jax.experimental.pallas import tpu as pltpu
from jax.experimental.pallas import tpu_sc as plsc

B, C, H, W = 32, 128, 100, 100
HW = H * W
V = B * HW
N = 65536
NC, NS, L = 2, 16, 16
NW = NC * NS
PER_W = N // NW
CHUNK = 128
NCHUNK = PER_W // CHUNK
CI = C // 2


def _transpose_body(x_ref, o_ref):
    o_ref[0] = x_ref[0].T


def _build_table(fused_grid):
    fg = fused_grid.reshape(B, C, HW)
    out = pl.pallas_call(
        _transpose_body,
        grid=(B,),
        in_specs=[pl.BlockSpec((1, C, HW), lambda b: (b, 0, 0))],
        out_specs=pl.BlockSpec((1, HW, C), lambda b: (b, 0, 0)),
        out_shape=jax.ShapeDtypeStruct((B, HW, C), jnp.float32),
    )(fg)
    return out.reshape(V, C)


_sc_mesh = plsc.VectorSubcoreMesh(core_axis_name="c", subcore_axis_name="s")


@functools.partial(
    pl.kernel,
    mesh=_sc_mesh,
    out_type=jax.ShapeDtypeStruct((N, C), jnp.float32),
    scratch_types=[
        pltpu.VMEM((PER_W,), jnp.int32),
        pltpu.VMEM((CHUNK, C), jnp.float32),
        pltpu.VMEM((CHUNK, C), jnp.float32),
        pltpu.VMEM((CHUNK, C), jnp.float32),
        pltpu.VMEM((CHUNK, C), jnp.float32),
        pltpu.SemaphoreType.DMA,
        pltpu.SemaphoreType.DMA,
        pltpu.SemaphoreType.DMA,
        pltpu.SemaphoreType.DMA,
        pltpu.SemaphoreType.DMA,
        pltpu.SemaphoreType.DMA,
    ],
)
def _sc_gather(table_hbm, idx_hbm, agent_hbm, out_hbm,
               idx_v, rows0, rows1, agn0, agn1,
               g0s, g1s, a0s, a1s, o0s, o1s):
    wid = lax.axis_index("c") * NS + lax.axis_index("s")
    base = wid * PER_W
    pltpu.sync_copy(idx_hbm.at[pl.ds(base, PER_W)], idx_v)

    def _add(rows_v, agn_v):
        @pl.loop(0, C, step=L)
        def _lane(j):
            for r in range(CHUNK):
                rows_v[r, pl.ds(j, L)] = (
                    rows_v[r, pl.ds(j, L)] + agn_v[r, pl.ds(j, L)])

    @pl.loop(0, NCHUNK, step=2)
    def _pair(kk):
        off0 = base + kk * CHUNK
        off1 = off0 + CHUNK
        g0 = pltpu.async_copy(
            table_hbm.at[idx_v.at[pl.ds(kk * CHUNK, CHUNK)]], rows0, g0s)
        a0 = pltpu.async_copy(agent_hbm.at[pl.ds(off0, CHUNK)], agn0, a0s)
        g1 = pltpu.async_copy(
            table_hbm.at[idx_v.at[pl.ds(kk * CHUNK + CHUNK, CHUNK)]],
            rows1, g1s)
        a1 = pltpu.async_copy(agent_hbm.at[pl.ds(off1, CHUNK)], agn1, a1s)
        g0.wait()
        a0.wait()
        _add(rows0, agn0)
        o0 = pltpu.async_copy(rows0, out_hbm.at[pl.ds(off0, CHUNK)], o0s)
        g1.wait()
        a1.wait()
        _add(rows1, agn1)
        o1 = pltpu.async_copy(rows1, out_hbm.at[pl.ds(off1, CHUNK)], o1s)
        o0.wait()
        o1.wait()


def _add_body(g_ref, a_ref, o_ref):
    o_ref[...] = g_ref[...] + a_ref[...]


def _tc_add(g, a):
    blk = 4096
    return pl.pallas_call(
        _add_body,
        grid=(N // blk,),
        in_specs=[pl.BlockSpec((blk, C), lambda i: (i, 0)),
                  pl.BlockSpec((blk, C), lambda i: (i, 0))],
        out_specs=pl.BlockSpec((blk, C), lambda i: (i, 0)),
        out_shape=jax.ShapeDtypeStruct((N, C), jnp.float32),
    )(g, a)


def kernel(fused_grid, agent_encodings, fetch_coordinates):
    table = _build_table(fused_grid)
    return _sc_gather(table, fetch_coordinates, agent_encodings)


# speedup vs baseline: 1.1897x; 1.0515x over previous
import functools

import jax
import jax.numpy as jnp
from jax import lax
from jax.experimental import pallas as pl
from jax.experimental.pallas import tpu as pltpu
from jax.experimental.pallas import tpu_sc as plsc

B, C, H, W = 32, 128, 100, 100
HW = H * W
V = B * HW
N = 65536
NC, NS, L = 2, 16, 16
NW = NC * NS
PER_W = N // NW
CHUNK = 128
NCHUNK = PER_W // CHUNK
CI = C // 2


def _transpose_body(x_ref, o_ref):
    o_ref[0] = x_ref[0].T


def _build_table(fused_grid):
    fg = fused_grid.reshape(B, C, HW)
    out = pl.pallas_call(
        _transpose_body,
        grid=(B,),
        in_specs=[pl.BlockSpec((1, C, HW), lambda b: (b, 0, 0))],
        out_specs=pl.BlockSpec((1, HW, C), lambda b: (b, 0, 0)),
        out_shape=jax.ShapeDtypeStruct((B, HW, C), jnp.float32),
    )(fg)
    return out.reshape(V, C)


_sc_mesh = plsc.VectorSubcoreMesh(core_axis_name="c", subcore_axis_name="s")


@functools.partial(
    pl.kernel,
    mesh=_sc_mesh,
    out_type=jax.ShapeDtypeStruct((N, C), jnp.float32),
    scratch_types=[
        pltpu.VMEM((PER_W,), jnp.int32),
        pltpu.VMEM((CHUNK, C), jnp.float32),
        pltpu.SemaphoreType.DMA,
    ],
)
def _sc_gather(table_hbm, idx_hbm, out_hbm, idx_v, rows_v, gsem):
    wid = lax.axis_index("c") * NS + lax.axis_index("s")
    base = wid * PER_W
    pltpu.sync_copy(idx_hbm.at[pl.ds(base, PER_W)], idx_v)

    @pl.loop(0, NCHUNK)
    def _chunk(k):
        off = base + k * CHUNK
        pltpu.async_copy(
            table_hbm.at[idx_v.at[pl.ds(k * CHUNK, CHUNK)]], rows_v,
            gsem).wait()
        pltpu.sync_copy(rows_v, out_hbm.at[pl.ds(off, CHUNK)])


def _add_body(g_ref, a_ref, o_ref):
    o_ref[...] = g_ref[...] + a_ref[...]


def _tc_add(g, a):
    blk = 4096
    return pl.pallas_call(
        _add_body,
        grid=(N // blk,),
        in_specs=[pl.BlockSpec((blk, C), lambda i: (i, 0)),
                  pl.BlockSpec((blk, C), lambda i: (i, 0))],
        out_specs=pl.BlockSpec((blk, C), lambda i: (i, 0)),
        out_shape=jax.ShapeDtypeStruct((N, C), jnp.float32),
    )(g, a)


def kernel(fused_grid, agent_encodings, fetch_coordinates):
    table = _build_table(fused_grid)
    gathered = _sc_gather(table, fetch_coordinates)
    return _tc_add(gathered, agent_encodings)


# speedup vs baseline: 1.1923x; 1.0022x over previous
import functools

import jax
import jax.numpy as jnp
from jax import lax
from jax.experimental import pallas as pl
from jax.experimental.pallas import tpu as pltpu
from jax.experimental.pallas import tpu_sc as plsc

B, C, H, W = 32, 128, 100, 100
HW = H * W
V = B * HW
N = 65536
NC, NS, L = 2, 16, 16
NW = NC * NS
PER_W = N // NW
CHUNK = 128
NCHUNK = PER_W // CHUNK


def _transpose_body(x_ref, o_ref):
    o_ref[0] = x_ref[0].T


def _build_table(fused_grid):
    fg = fused_grid.reshape(B, C, HW)
    out = pl.pallas_call(
        _transpose_body,
        grid=(B,),
        in_specs=[pl.BlockSpec((1, C, HW), lambda b: (b, 0, 0))],
        out_specs=pl.BlockSpec((1, HW, C), lambda b: (b, 0, 0)),
        out_shape=jax.ShapeDtypeStruct((B, HW, C), jnp.float32),
    )(fg)
    return out.reshape(V, C)


_sc_mesh = plsc.VectorSubcoreMesh(core_axis_name="c", subcore_axis_name="s")


@functools.partial(
    pl.kernel,
    mesh=_sc_mesh,
    out_type=jax.ShapeDtypeStruct((N, C), jnp.float32),
    scratch_types=[
        pltpu.VMEM((PER_W,), jnp.int32),
        pltpu.VMEM((CHUNK, C), jnp.float32),
        pltpu.SemaphoreType.DMA,
    ],
)
def _sc_gather(table_hbm, idx_hbm, out_hbm, idx_v, rows_v, gsem):
    wid = lax.axis_index("c") * NS + lax.axis_index("s")
    base = wid * PER_W
    pltpu.sync_copy(idx_hbm.at[pl.ds(base, PER_W)], idx_v)

    @pl.loop(0, NCHUNK)
    def _chunk(k):
        off = base + k * CHUNK
        pltpu.async_copy(
            table_hbm.at[idx_v.at[pl.ds(k * CHUNK, CHUNK)]], rows_v,
            gsem).wait()
        pltpu.sync_copy(rows_v, out_hbm.at[pl.ds(off, CHUNK)])


def _add_body(g_ref, a_ref, o_ref):
    o_ref[...] = g_ref[...] + a_ref[...]


def _tc_add(g, a):
    blk = 4096
    return pl.pallas_call(
        _add_body,
        grid=(N // blk,),
        in_specs=[pl.BlockSpec((blk, C), lambda i: (i, 0)),
                  pl.BlockSpec((blk, C), lambda i: (i, 0))],
        out_specs=pl.BlockSpec((blk, C), lambda i: (i, 0)),
        out_shape=jax.ShapeDtypeStruct((N, C), jnp.float32),
    )(g, a)


def kernel(fused_grid, agent_encodings, fetch_coordinates):
    table = _build_table(fused_grid)
    gathered = _sc_gather(table, fetch_coordinates)
    return _tc_add(gathered, agent_encodings)


# speedup vs baseline: 1.2198x; 1.0231x over previous
import functools

import jax
import jax.numpy as jnp
from jax import lax
from jax.experimental import pallas as pl
from jax.experimental.pallas import tpu as pltpu
from jax.experimental.pallas import tpu_sc as plsc

B, C, H, W = 32, 128, 100, 100
HW = H * W
V = B * HW
N = 65536
NC, NS, L = 2, 16, 16
NW = NC * NS
PER_W = N // NW
CHUNK = 128
NCHUNK = PER_W // CHUNK


def _transpose_body(x_ref, o_ref):
    o_ref[0] = x_ref[0].T


def _build_table(fused_grid):
    fg = fused_grid.reshape(B, C, HW)
    out = pl.pallas_call(
        _transpose_body,
        grid=(B,),
        in_specs=[pl.BlockSpec((1, C, HW), lambda b: (b, 0, 0))],
        out_specs=pl.BlockSpec((1, HW, C), lambda b: (b, 0, 0)),
        out_shape=jax.ShapeDtypeStruct((B, HW, C), jnp.float32),
    )(fg)
    return out.reshape(V, C)


_sc_mesh = plsc.VectorSubcoreMesh(core_axis_name="c", subcore_axis_name="s")


@functools.partial(
    pl.kernel,
    mesh=_sc_mesh,
    out_type=jax.ShapeDtypeStruct((N, C), jnp.float32),
    scratch_types=[
        pltpu.VMEM((PER_W,), jnp.int32),
        pltpu.VMEM((CHUNK, C), jnp.float32),
        pltpu.VMEM((CHUNK, C), jnp.float32),
        pltpu.SemaphoreType.DMA,
        pltpu.SemaphoreType.DMA,
        pltpu.SemaphoreType.DMA,
        pltpu.SemaphoreType.DMA,
    ],
)
def _sc_gather(table_hbm, idx_hbm, out_hbm, idx_v, rows0, rows1,
               g0s, g1s, o0s, o1s):
    wid = lax.axis_index("c") * NS + lax.axis_index("s")
    base = wid * PER_W
    pltpu.sync_copy(idx_hbm.at[pl.ds(base, PER_W)], idx_v)

    @pl.loop(0, NCHUNK, step=2)
    def _pair(kk):
        off0 = base + kk * CHUNK
        off1 = off0 + CHUNK
        g0 = pltpu.async_copy(
            table_hbm.at[idx_v.at[pl.ds(kk * CHUNK, CHUNK)]], rows0, g0s)
        g1 = pltpu.async_copy(
            table_hbm.at[idx_v.at[pl.ds(kk * CHUNK + CHUNK, CHUNK)]],
            rows1, g1s)
        g0.wait()
        o0 = pltpu.async_copy(rows0, out_hbm.at[pl.ds(off0, CHUNK)], o0s)
        g1.wait()
        o1 = pltpu.async_copy(rows1, out_hbm.at[pl.ds(off1, CHUNK)], o1s)
        o0.wait()
        o1.wait()


def _add_body(g_ref, a_ref, o_ref):
    o_ref[...] = g_ref[...] + a_ref[...]


def _tc_add(g, a):
    blk = 4096
    return pl.pallas_call(
        _add_body,
        grid=(N // blk,),
        in_specs=[pl.BlockSpec((blk, C), lambda i: (i, 0)),
                  pl.BlockSpec((blk, C), lambda i: (i, 0))],
        out_specs=pl.BlockSpec((blk, C), lambda i: (i, 0)),
        out_shape=jax.ShapeDtypeStruct((N, C), jnp.float32),
    )(g, a)


def kernel(fused_grid, agent_encodings, fetch_coordinates):
    table = _build_table(fused_grid)
    gathered = _sc_gather(table, fetch_coordinates)
    return _tc_add(gathered, agent_encodings)


# speedup vs baseline: 1.2236x; 1.0031x over previous
import functools

import jax
import jax.numpy as jnp
from jax import lax
from jax.experimental import pallas as pl
from jax.experimental.pallas import tpu as pltpu
from jax.experimental.pallas import tpu_sc as plsc

B, C, H, W = 32, 128, 100, 100
HW = H * W
V = B * HW
N = 65536
NC, NS, L = 2, 16, 16
NW = NC * NS
PER_W = N // NW
CHUNK = 128
NCHUNK = PER_W // CHUNK


def _transpose_body(x_ref, o_ref):
    o_ref[0] = x_ref[0].T


def _build_table(fused_grid):
    fg = fused_grid.reshape(B, C, HW)
    out = pl.pallas_call(
        _transpose_body,
        grid=(B,),
        in_specs=[pl.BlockSpec((1, C, HW), lambda b: (b, 0, 0))],
        out_specs=pl.BlockSpec((1, HW, C), lambda b: (b, 0, 0)),
        out_shape=jax.ShapeDtypeStruct((B, HW, C), jnp.float32),
    )(fg)
    return out.reshape(V, C)


_sc_mesh = plsc.VectorSubcoreMesh(core_axis_name="c", subcore_axis_name="s")


@functools.partial(
    pl.kernel,
    mesh=_sc_mesh,
    out_type=jax.ShapeDtypeStruct((N, C), jnp.float32),
    scratch_types=[
        pltpu.VMEM((PER_W,), jnp.int32),
        pltpu.VMEM((4, CHUNK, C), jnp.float32),
        pltpu.SemaphoreType.DMA,
        pltpu.SemaphoreType.DMA,
        pltpu.SemaphoreType.DMA,
        pltpu.SemaphoreType.DMA,
        pltpu.SemaphoreType.DMA,
        pltpu.SemaphoreType.DMA,
        pltpu.SemaphoreType.DMA,
        pltpu.SemaphoreType.DMA,
    ],
)
def _sc_gather(table_hbm, idx_hbm, out_hbm, idx_v, rows,
               g0s, g1s, g2s, g3s, o0s, o1s, o2s, o3s):
    wid = lax.axis_index("c") * NS + lax.axis_index("s")
    base = wid * PER_W
    pltpu.sync_copy(idx_hbm.at[pl.ds(base, PER_W)], idx_v)
    gsems = (g0s, g1s, g2s, g3s)
    osems = (o0s, o1s, o2s, o3s)

    @pl.loop(0, NCHUNK, step=4)
    def _quad(kk):
        gs = [pltpu.async_copy(
                  table_hbm.at[idx_v.at[pl.ds(kk * CHUNK + p * CHUNK,
                                              CHUNK)]],
                  rows.at[p], gsems[p])
              for p in range(4)]
        os = []
        for p in range(4):
            gs[p].wait()
            os.append(pltpu.async_copy(
                rows.at[p],
                out_hbm.at[pl.ds(base + kk * CHUNK + p * CHUNK, CHUNK)],
                osems[p]))
        for p in range(4):
            os[p].wait()


def _add_body(g_ref, a_ref, o_ref):
    o_ref[...] = g_ref[...] + a_ref[...]


def _tc_add(g, a):
    blk = 4096
    return pl.pallas_call(
        _add_body,
        grid=(N // blk,),
        in_specs=[pl.BlockSpec((blk, C), lambda i: (i, 0)),
                  pl.BlockSpec((blk, C), lambda i: (i, 0))],
        out_specs=pl.BlockSpec((blk, C), lambda i: (i, 0)),
        out_shape=jax.ShapeDtypeStruct((N, C), jnp.float32),
    )(g, a)


def kernel(fused_grid, agent_encodings, fetch_coordinates):
    table = _build_table(fused_grid)
    gathered = _sc_gather(table, fetch_coordinates)
    return _tc_add(gathered, agent_encodings)
